# baseline (speedup 1.0000x reference)
import contextlib
import functools

import jax
import jax.numpy as jnp
import numpy as np
from jax.experimental import pallas as pl
from jax.experimental.pallas import tpu as pltpu

_MASK_PROB = 0.2
_MASK_SPAN = 10
_W = 64

_START_CACHE = {}


def _get_starts(B, T):
    if (B, T) not in _START_CACHE:
        num_spans = max(1, int(_MASK_PROB * (T / _MASK_SPAN)))
        max_start = max(1, T - _MASK_SPAN)
        try:
            dev_ctx = jax.default_device(jax.local_devices(backend="cpu")[0])
        except Exception:
            dev_ctx = contextlib.nullcontext()
        with jax.ensure_compile_time_eval(), dev_ctx:
            keys = jax.random.split(jax.random.key(42), B)
            rows = [np.asarray(jax.random.permutation(k, max_start))[:num_spans]
                    for k in keys]
        _START_CACHE[(B, T)] = np.stack(rows).astype(np.int32)
    return _START_CACHE[(B, T)]


def _runs_and_gaps(starts_row, T):
    ss = np.sort(starts_row)
    runs = []
    cs, ce = int(ss[0]), min(T, int(ss[0]) + _MASK_SPAN)
    for s in ss[1:]:
        s = int(s)
        e = min(T, s + _MASK_SPAN)
        if s <= ce:
            ce = max(ce, e)
        else:
            runs.append((cs, ce))
            cs, ce = s, e
    runs.append((cs, ce))
    gaps, prev = [], 0
    for (s, e) in runs:
        if s > prev:
            gaps.append((prev, s))
        prev = e
    if prev < T:
        gaps.append((prev, T))
    return runs, gaps


def _body(span, jobs, maxrun, D, starts_col_ref, emb_ref, z_ref, out_ref,
          m_ref, emb_buf, sems, esem):
    for j in range(maxrun):
        pltpu.make_async_copy(emb_ref, emb_buf.at[pl.ds(j * D, D)],
                              esem).start()
    for j in range(maxrun):
        pltpu.make_async_copy(emb_ref, emb_buf.at[pl.ds(j * D, D)],
                              esem).wait()
    descs = []
    for (kind, o, n) in jobs:
        i = len(descs)
        if kind == 0:
            d = pltpu.make_async_copy(
                z_ref.at[pl.ds(o, n)], out_ref.at[pl.ds(o, n)],
                sems.at[i % _W])
        else:
            d = pltpu.make_async_copy(
                emb_buf.at[pl.ds(0, n)], out_ref.at[pl.ds(o, n)],
                sems.at[i % _W])
        d.start()
        if i >= _W:
            descs[i - _W].wait()
        descs.append(d)
    s_padc = starts_col_ref.shape[1]
    t_full = m_ref.shape[2]
    for b in range(m_ref.shape[0]):
        stc = starts_col_ref[b]
        tic = jax.lax.broadcasted_iota(jnp.int32, (s_padc, t_full), 1)
        hitc = (tic >= stc) & (tic < stc + span)
        m_ref[b] = jnp.any(hitc, axis=0, keepdims=True)
    for d in descs[-_W:]:
        d.wait()


def kernel(z_t, mask_emb):
    B, T, D = z_t.shape
    starts = _get_starts(B, T)
    S = starts.shape[1]
    jobs = []
    maxrun = 1
    for b in range(B):
        runs, gaps = _runs_and_gaps(starts[b], T)
        maxrun = max(maxrun, max(e - s for s, e in runs))
        jobs += [(0, (b * T + s) * D, (e - s) * D) for s, e in gaps]
        jobs += [(1, (b * T + s) * D, (e - s) * D) for s, e in runs]
    S_PADC = -(-S // 8) * 8
    starts_col = np.full((B, S_PADC, 1), T, dtype=np.int32)
    starts_col[:, :S, 0] = starts
    starts_col = jnp.asarray(starts_col)

    out, mask = pl.pallas_call(
        functools.partial(_body, _MASK_SPAN, tuple(jobs), maxrun, D),
        in_specs=[
            pl.BlockSpec((B, S_PADC, 1), lambda: (0, 0, 0)),
            pl.BlockSpec(memory_space=pl.ANY),
            pl.BlockSpec(memory_space=pl.ANY),
        ],
        out_specs=[
            pl.BlockSpec(memory_space=pl.ANY),
            pl.BlockSpec((B, 1, T), lambda: (0, 0, 0)),
        ],
        out_shape=[
            jax.ShapeDtypeStruct((B * T * D,), z_t.dtype),
            jax.ShapeDtypeStruct((B, 1, T), jnp.bool_),
        ],
        scratch_shapes=[
            pltpu.VMEM((maxrun * D,), jnp.float32),
            pltpu.SemaphoreType.DMA((_W,)),
            pltpu.SemaphoreType.DMA,
        ],
        compiler_params=pltpu.CompilerParams(
            vmem_limit_bytes=100 * 1024 * 1024,
        ),
    )(starts_col, mask_emb, z_t.reshape(-1))
    return out.reshape(B, T, D), mask.reshape(B, T)


# speedup vs baseline: 42.0245x; 42.0245x over previous
import contextlib
import functools

import jax
import jax.numpy as jnp
import numpy as np
from jax.experimental import pallas as pl
from jax.experimental.pallas import tpu as pltpu

_MASK_PROB = 0.2
_MASK_SPAN = 10

_START_CACHE = {}


def _get_starts(B, T):
    if (B, T) not in _START_CACHE:
        num_spans = max(1, int(_MASK_PROB * (T / _MASK_SPAN)))
        max_start = max(1, T - _MASK_SPAN)
        try:
            dev_ctx = jax.default_device(jax.local_devices(backend="cpu")[0])
        except Exception:
            dev_ctx = contextlib.nullcontext()
        with jax.ensure_compile_time_eval(), dev_ctx:
            keys = jax.random.split(jax.random.key(42), B)
            rows = [np.asarray(jax.random.permutation(k, max_start))[:num_spans]
                    for k in keys]
        _START_CACHE[(B, T)] = np.stack(rows).astype(np.int32)
    return _START_CACHE[(B, T)]


def _mask_body(span, starts_ref, starts_col_ref, emb_ref, z_ref, out_ref, m_ref):
    t_blk = pl.program_id(1)
    tt = out_ref.shape[1]
    s_pad = starts_ref.shape[2]
    st = starts_ref[0]
    ti = jax.lax.broadcasted_iota(jnp.int32, (tt, s_pad), 0) + t_blk * tt
    hit = (ti >= st) & (ti < st + span)
    mrow = jnp.any(hit, axis=1, keepdims=True)
    s_padc = starts_col_ref.shape[1]
    stc = starts_col_ref[0]
    tic = jax.lax.broadcasted_iota(jnp.int32, (s_padc, tt), 1) + t_blk * tt
    hitc = (tic >= stc) & (tic < stc + span)
    m_ref[0] = jnp.any(hitc, axis=0, keepdims=True)
    out_ref[0] = jnp.where(mrow, emb_ref[...], z_ref[0])


def kernel(z_t, mask_emb):
    B, T, D = z_t.shape
    starts = _get_starts(B, T)
    S = starts.shape[1]
    S_PAD = -(-S // 128) * 128
    starts3 = np.full((B, 1, S_PAD), T, dtype=np.int32)
    starts3[:, 0, :S] = starts
    starts3 = jnp.asarray(starts3)
    S_PADC = -(-S // 8) * 8
    starts_col = np.full((B, S_PADC, 1), T, dtype=np.int32)
    starts_col[:, :S, 0] = starts
    starts_col = jnp.asarray(starts_col)

    TT = 2048
    grid = (B, T // TT)
    out, mask = pl.pallas_call(
        functools.partial(_mask_body, _MASK_SPAN),
        grid=grid,
        in_specs=[
            pl.BlockSpec((1, 1, S_PAD), lambda b, t: (b, 0, 0)),
            pl.BlockSpec((1, S_PADC, 1), lambda b, t: (b, 0, 0)),
            pl.BlockSpec((1, D), lambda b, t: (0, 0)),
            pl.BlockSpec((1, TT, D), lambda b, t: (b, t, 0)),
        ],
        out_specs=[
            pl.BlockSpec((1, TT, D), lambda b, t: (b, t, 0)),
            pl.BlockSpec((1, 1, TT), lambda b, t: (b, 0, t)),
        ],
        out_shape=[
            jax.ShapeDtypeStruct((B, T, D), z_t.dtype),
            jax.ShapeDtypeStruct((B, 1, T), jnp.bool_),
        ],
        compiler_params=pltpu.CompilerParams(
            dimension_semantics=("parallel", "parallel"),
            vmem_limit_bytes=100 * 1024 * 1024,
        ),
    )(starts3, starts_col, mask_emb.reshape(1, D), z_t)
    return out, mask.reshape(B, T)
